# Initial kernel scaffold; baseline (speedup 1.0000x reference)
import jax
import jax.numpy as jnp
from jax.experimental import pallas as pl


def kernel(node_embs, edge_embs, rel_embs, edge_index, W_n1, W_s1, g1, b1, W_e1, be1, W_n2, W_s2, g2, b2, W_e2, be2):
    raise NotImplementedError("write your pallas kernel here")


# speedup vs baseline: 1.3724x; 1.3724x over previous
import functools

import jax
import jax.numpy as jnp
from jax import lax
from jax.experimental import pallas as pl
from jax.experimental.pallas import tpu as pltpu
from jax.experimental.pallas import tpu_sc as plsc

N = 10000
E = 160000
D = 256
H = 128
NS = 16
CH = 128
ROWS = E // CH
RPT = 80
ROWS_P = RPT * NS
NTRASH = 16
NA = N + NTRASH
ACC_PT = 624
IB = 16
SUB = 64
NSUB = CH // SUB
BN = 1000
BE = 1000


def _make_sc_agg(edge_wide):
    out_type = [jax.ShapeDtypeStruct((2 * N, H), jnp.float32)]

    scratch = [
        pltpu.VMEM_SHARED((NA, H), jnp.float32),
        pltpu.VMEM((IB, NSUB, SUB), jnp.int32),
        pltpu.VMEM((IB, NSUB, SUB), jnp.int32),
        pltpu.VMEM((SUB, H), jnp.float32),
        pltpu.VMEM((SUB, H), jnp.float32),
        pltpu.SemaphoreType.DMA,
    ]

    mesh = plsc.VectorSubcoreMesh(core_axis_name="c", subcore_axis_name="s")

    def body(node_t, edge_t, src2_r, dst_r, zacc, agg_out,
             acc, idx_s, idx_d, gath, edgeb, sem):
        c = lax.axis_index("c")
        s = lax.axis_index("s")
        nbase = s * ACC_PT
        tail0 = NS * ACC_PT
        tail_init = NA - tail0

        pltpu.sync_copy(zacc.at[pl.ds(nbase, ACC_PT)],
                        acc.at[pl.ds(nbase, ACC_PT)])

        @pl.when(s == NS - 1)
        def _():
            pltpu.sync_copy(zacc.at[pl.ds(tail0, tail_init)],
                            acc.at[pl.ds(tail0, tail_init)])

        lo = s * RPT
        plsc.subcore_barrier()

        def do_batch(jj, carry):
            b, j = carry
            row = jnp.minimum(lo + b * IB + j, ROWS - 1)
            for q in range(NSUB):
                pltpu.async_copy(node_t.at[idx_s.at[j, q]], gath, sem).wait()
                e0 = row * CH + q * SUB
                if edge_wide:
                    pltpu.sync_copy(
                        edge_t.at[pl.ds(e0, SUB), pl.ds(c * H, H)], edgeb)
                else:
                    pltpu.sync_copy(edge_t.at[pl.ds(c * E + e0, SUB)], edgeb)

                def mulrow(r, carry2):
                    for p in range(H // 16):
                        sl = pl.ds(p * 16, 16)
                        gath[r, sl] = gath[r, sl] * edgeb[r, sl]
                    return carry2
                lax.fori_loop(0, SUB, mulrow, 0, unroll=2)

                pltpu.sync_copy(gath, acc.at[idx_d.at[j, q]], add=True)
            return (b, j + 1)

        def do_block(b, carry):
            pltpu.sync_copy(src2_r.at[pl.ds(c * ROWS_P + lo + b * IB, IB)],
                            idx_s)
            pltpu.sync_copy(dst_r.at[pl.ds(lo + b * IB, IB)], idx_d)
            lax.fori_loop(0, IB, do_batch, (b, 0))
            return carry

        lax.fori_loop(0, RPT // IB, do_block, 0)
        plsc.subcore_barrier()

        pltpu.sync_copy(acc.at[pl.ds(nbase, ACC_PT)],
                        agg_out.at[pl.ds(c * N + nbase, ACC_PT)])

        @pl.when(s == NS - 1)
        def _():
            pltpu.sync_copy(acc.at[pl.ds(tail0, N - tail0)],
                            agg_out.at[pl.ds(c * N + tail0, N - tail0)])

    return pl.kernel(body, out_type=out_type, mesh=mesh,
                     scratch_types=scratch)


def _make_sc_deg():
    RPT_D = ROWS_P // (2 * NS)
    IB_D = 8

    scratch = [
        pltpu.VMEM_SHARED((NA, H), jnp.float32),
        pltpu.VMEM((IB_D, NSUB, SUB), jnp.int32),
        pltpu.VMEM((SUB, H), jnp.float32),
    ]
    mesh = plsc.VectorSubcoreMesh(core_axis_name="c", subcore_axis_name="s")

    def body(dst_r, zacc, deg_out, dacc, idx_d, ones_v):
        c = lax.axis_index("c")
        s = lax.axis_index("s")
        nbase = s * ACC_PT
        tail0 = NS * ACC_PT
        tail_init = NA - tail0

        pltpu.sync_copy(zacc.at[pl.ds(nbase, ACC_PT)],
                        dacc.at[pl.ds(nbase, ACC_PT)])

        @pl.when(s == NS - 1)
        def _():
            pltpu.sync_copy(zacc.at[pl.ds(tail0, tail_init)],
                            dacc.at[pl.ds(tail0, tail_init)])

        def _fill(i, carry):
            for p in range(H // 16):
                ones_v[i, pl.ds(p * 16, 16)] = jnp.full((16,), 1.0,
                                                        jnp.float32)
            return carry
        lax.fori_loop(0, SUB, _fill, 0)

        lo = c * (ROWS_P // 2) + s * RPT_D
        plsc.subcore_barrier()

        def do_batch(jj, carry):
            j = carry
            for q in range(NSUB):
                pltpu.sync_copy(ones_v, dacc.at[idx_d.at[j, q]], add=True)
            return j + 1

        def do_block(b, carry):
            pltpu.sync_copy(dst_r.at[pl.ds(lo + b * IB_D, IB_D)], idx_d)
            lax.fori_loop(0, IB_D, do_batch, 0)
            return carry

        lax.fori_loop(0, RPT_D // IB_D, do_block, 0)
        plsc.subcore_barrier()

        pltpu.sync_copy(dacc.at[pl.ds(nbase, ACC_PT)],
                        deg_out.at[pl.ds(c * N + nbase, ACC_PT)])

        @pl.when(s == NS - 1)
        def _():
            pltpu.sync_copy(dacc.at[pl.ds(tail0, N - tail0)],
                            deg_out.at[pl.ds(c * N + tail0, N - tail0)])

    return pl.kernel(body,
                     out_type=[jax.ShapeDtypeStruct((2 * N, H), jnp.float32)],
                     mesh=mesh, scratch_types=scratch)


@functools.lru_cache(maxsize=None)
def _sc_agg(edge_wide):
    return _make_sc_agg(edge_wide=edge_wide)


@functools.lru_cache(maxsize=None)
def _sc_deg():
    return _make_sc_deg()


def _dense_body(agg_ref, deg_ref, x_ref, wn_ref, ws_ref, h_ref, sums_ref):
    i = pl.program_id(0)
    aggf = jnp.concatenate([agg_ref[0], agg_ref[1]], axis=1)
    xf = jnp.concatenate([x_ref[0], x_ref[1]], axis=1)
    d = deg_ref[0][:, 0:1] + deg_ref[1][:, 0:1]
    r = 1.0 / jnp.maximum(d, 1.0)
    h = (jnp.dot(aggf * r, wn_ref[...], preferred_element_type=jnp.float32)
         + jnp.dot(xf, ws_ref[...], preferred_element_type=jnp.float32))
    h_ref[...] = h

    @pl.when(i == 0)
    def _():
        sums_ref[...] = jnp.zeros_like(sums_ref)

    sums_ref[0:1, :] += jnp.sum(h, axis=0, keepdims=True)
    sums_ref[1:2, :] += jnp.sum(h * h, axis=0, keepdims=True)


def _dense(agg_st, deg, x_st, wn, ws):
    return pl.pallas_call(
        _dense_body,
        grid=(N // BN,),
        in_specs=[
            pl.BlockSpec((2, BN, H), lambda i: (0, i, 0)),
            pl.BlockSpec((2, BN, H), lambda i: (0, i, 0)),
            pl.BlockSpec((2, BN, H), lambda i: (0, i, 0)),
            pl.BlockSpec((D, D), lambda i: (0, 0)),
            pl.BlockSpec((D, D), lambda i: (0, 0)),
        ],
        out_specs=[
            pl.BlockSpec((BN, D), lambda i: (i, 0)),
            pl.BlockSpec((8, D), lambda i: (0, 0)),
        ],
        out_shape=[
            jax.ShapeDtypeStruct((N, D), jnp.float32),
            jax.ShapeDtypeStruct((8, D), jnp.float32),
        ],
    )(agg_st, deg, x_st, wn, ws)


def _norm_body(h_ref, sums_ref, g_ref, b_ref, out_ref, *, stacked):
    m = sums_ref[0:1, :] / float(N)
    v = sums_ref[1:2, :] / float(N) - m * m
    inv = lax.rsqrt(v + 1e-5)
    y = (h_ref[...] - m) * (inv * g_ref[...]) + b_ref[...]
    y = jnp.maximum(y, 0.0)
    if stacked:
        out_ref[0] = y[:, :H]
        out_ref[1] = y[:, H:]
    else:
        out_ref[...] = y


def _norm(h, sums, g, b, stacked):
    if stacked:
        out_spec = pl.BlockSpec((2, BN, H), lambda i: (0, i, 0))
        out_shape = jax.ShapeDtypeStruct((2, N, H), jnp.float32)
    else:
        out_spec = pl.BlockSpec((BN, D), lambda i: (i, 0))
        out_shape = jax.ShapeDtypeStruct((N, D), jnp.float32)
    return pl.pallas_call(
        functools.partial(_norm_body, stacked=stacked),
        grid=(N // BN,),
        in_specs=[
            pl.BlockSpec((BN, D), lambda i: (i, 0)),
            pl.BlockSpec((8, D), lambda i: (0, 0)),
            pl.BlockSpec((1, D), lambda i: (0, 0)),
            pl.BlockSpec((1, D), lambda i: (0, 0)),
        ],
        out_specs=out_spec,
        out_shape=out_shape,
    )(h, sums, g, b)


def _edge_body(e_ref, w_ref, b_ref, out_ref):
    y = jnp.dot(e_ref[...], w_ref[...], preferred_element_type=jnp.float32)
    y = jnp.maximum(y + b_ref[...], 0.0)
    out_ref[0] = y[:, :H]
    out_ref[1] = y[:, H:]


def _edge_tf(edge, w, b):
    return pl.pallas_call(
        _edge_body,
        grid=(E // BE,),
        in_specs=[
            pl.BlockSpec((BE, D), lambda i: (i, 0)),
            pl.BlockSpec((D, D), lambda i: (0, 0)),
            pl.BlockSpec((1, D), lambda i: (0, 0)),
        ],
        out_specs=pl.BlockSpec((2, BE, H), lambda i: (0, i, 0)),
        out_shape=jax.ShapeDtypeStruct((2, E, H), jnp.float32),
    )(edge, w, b)


def _rel_body(rel_ref, w1_ref, b1_ref, w2_ref, b2_ref, out_ref):
    y = jnp.dot(rel_ref[...], w1_ref[...], preferred_element_type=jnp.float32)
    y = jnp.maximum(y + b1_ref[...], 0.0)
    y = jnp.dot(y, w2_ref[...], preferred_element_type=jnp.float32)
    out_ref[...] = jnp.maximum(y + b2_ref[...], 0.0)


def _rel_mlp(rel, w1, b1, w2, b2):
    return pl.pallas_call(
        _rel_body,
        out_shape=jax.ShapeDtypeStruct(rel.shape, jnp.float32),
    )(rel, w1, b1, w2, b2)


def kernel(node_embs, edge_embs, rel_embs, edge_index,
           W_n1, W_s1, g1, b1, W_e1, be1,
           W_n2, W_s2, g2, b2, W_e2, be2):
    src = edge_index[0]
    dst = edge_index[1]

    node_st = jnp.stack([node_embs[:, :H], node_embs[:, H:]])
    npad = ROWS_P * CH - E
    src_pad = jnp.concatenate([src, jnp.zeros((npad,), jnp.int32)])
    src2_r = jnp.concatenate([src_pad, src_pad + N]).reshape(
        2 * ROWS_P, NSUB, SUB)
    dst_pad = jnp.concatenate(
        [dst, N + (jnp.arange(npad, dtype=jnp.int32) % NTRASH)])
    dst_r = dst_pad.reshape(ROWS_P, NSUB, SUB)
    zacc = jnp.zeros((NA, H), jnp.float32)

    (deg2,) = _sc_deg()(dst_r, zacc)
    deg = deg2.reshape(2, N, H)
    (agg1,) = _sc_agg(True)(node_st.reshape(2 * N, H), edge_embs,
                            src2_r, dst_r, zacc)
    h1, sums1 = _dense(agg1.reshape(2, N, H), deg, node_st, W_n1, W_s1)
    h1r_st = _norm(h1, sums1, g1.reshape(1, D), b1.reshape(1, D), stacked=True)
    e1r_st = _edge_tf(edge_embs, W_e1, be1.reshape(1, D))

    (agg2,) = _sc_agg(False)(h1r_st.reshape(2 * N, H),
                             e1r_st.reshape(2 * E, H),
                             src2_r, dst_r, zacc)
    h2, sums2 = _dense(agg2.reshape(2, N, H), deg, h1r_st, W_n2, W_s2)
    nodes_out = _norm(h2, sums2, g2.reshape(1, D), b2.reshape(1, D),
                      stacked=False)

    r = _rel_mlp(rel_embs, W_e1, be1.reshape(1, D), W_e2, be2.reshape(1, D))
    return (nodes_out, r)


# speedup vs baseline: 1.8599x; 1.3552x over previous
import functools

import jax
import jax.numpy as jnp
from jax import lax
from jax.experimental import pallas as pl
from jax.experimental.pallas import tpu as pltpu
from jax.experimental.pallas import tpu_sc as plsc

N = 10000
E = 160000
D = 256
H = 128
NS = 16
CH = 128
ROWS = E // CH
RPT = 80
ROWS_P = RPT * NS
NTRASH = 16
NA = N + NTRASH
ACC_PT = 624
IB = 16
SUB = 64
NSUB = CH // SUB
BN = 1000
BE = 1000


def _make_sc_agg(edge_wide):
    out_type = [jax.ShapeDtypeStruct((2 * N, H), jnp.float32)]

    scratch = [
        pltpu.VMEM_SHARED((NA, H), jnp.float32),
        pltpu.VMEM((IB, NSUB, SUB), jnp.int32),
        pltpu.VMEM((IB, NSUB, SUB), jnp.int32),
        pltpu.VMEM((SUB, H), jnp.float32),
        pltpu.VMEM((SUB, H), jnp.float32),
        pltpu.VMEM((SUB, H), jnp.float32),
        pltpu.VMEM((SUB, H), jnp.float32),
        pltpu.SemaphoreType.DMA,
        pltpu.SemaphoreType.DMA,
        pltpu.SemaphoreType.DMA,
        pltpu.SemaphoreType.DMA,
    ]

    mesh = plsc.VectorSubcoreMesh(core_axis_name="c", subcore_axis_name="s")

    def body(node_t, edge_t, src2_r, dst_r, zacc, agg_out,
             acc, idx_s, idx_d, gath_a, gath_b, edge_a, edge_b,
             sem_a, sem_b, ssem_a, ssem_b):
        c = lax.axis_index("c")
        s = lax.axis_index("s")
        nbase = s * ACC_PT
        tail0 = NS * ACC_PT
        tail_init = NA - tail0

        pltpu.sync_copy(zacc.at[pl.ds(nbase, ACC_PT)],
                        acc.at[pl.ds(nbase, ACC_PT)])

        @pl.when(s == NS - 1)
        def _():
            pltpu.sync_copy(zacc.at[pl.ds(tail0, tail_init)],
                            acc.at[pl.ds(tail0, tail_init)])

        lo = s * RPT

        def edge_src(row, q):
            e0 = row * CH + q * SUB
            if edge_wide:
                return edge_t.at[pl.ds(e0, SUB), pl.ds(c * H, H)]
            return edge_t.at[pl.ds(c * E + e0, SUB)]

        def issue_in(j, q, gbuf, ebuf, sem, blk0):
            row = jnp.minimum(blk0 + j, ROWS - 1)
            pltpu.async_copy(node_t.at[idx_s.at[j, q]], gbuf, sem)
            pltpu.async_copy(edge_src(row, q), ebuf, sem)

        def wait_in(j, q, gbuf, ebuf, sem, blk0):
            row = jnp.minimum(blk0 + j, ROWS - 1)
            pltpu.make_async_copy(node_t.at[idx_s.at[j, q]], gbuf, sem).wait()
            pltpu.make_async_copy(edge_src(row, q), ebuf, sem).wait()

        def mul(gbuf, ebuf):
            def mulrow(r, carry2):
                for p in range(H // 16):
                    sl = pl.ds(p * 16, 16)
                    gbuf[r, sl] = gbuf[r, sl] * ebuf[r, sl]
                return carry2
            lax.fori_loop(0, SUB, mulrow, 0, unroll=2)

        def scat(j, q, gbuf, sem):
            pltpu.async_copy(gbuf, acc.at[idx_d.at[j, q]], sem, add=True)

        def scat_wait(j, q, gbuf, sem):
            pltpu.make_async_copy(gbuf, acc.at[idx_d.at[j, q]], sem).wait()

        plsc.subcore_barrier()

        for blk in range(RPT // IB):
            blk0 = lo + blk * IB
            pltpu.sync_copy(src2_r.at[pl.ds(c * ROWS_P + blk0, IB)], idx_s)
            pltpu.sync_copy(dst_r.at[pl.ds(blk0, IB)], idx_d)
            issue_in(0, 0, gath_a, edge_a, sem_a, blk0)

            def pair(j, carry):
                wait_in(j, 0, gath_a, edge_a, sem_a, blk0)

                @pl.when(j > 0)
                def _():
                    scat_wait(j - 1, 1, gath_b, ssem_b)
                issue_in(j, 1, gath_b, edge_b, sem_b, blk0)
                mul(gath_a, edge_a)
                scat(j, 0, gath_a, ssem_a)
                wait_in(j, 1, gath_b, edge_b, sem_b, blk0)
                mul(gath_b, edge_b)
                scat_wait(j, 0, gath_a, ssem_a)

                @pl.when(j < IB - 1)
                def _():
                    issue_in(j + 1, 0, gath_a, edge_a, sem_a, blk0)
                scat(j, 1, gath_b, ssem_b)
                return carry

            lax.fori_loop(0, IB, pair, 0)
            scat_wait(IB - 1, 1, gath_b, ssem_b)

        plsc.subcore_barrier()

        pltpu.sync_copy(acc.at[pl.ds(nbase, ACC_PT)],
                        agg_out.at[pl.ds(c * N + nbase, ACC_PT)])

        @pl.when(s == NS - 1)
        def _():
            pltpu.sync_copy(acc.at[pl.ds(tail0, N - tail0)],
                            agg_out.at[pl.ds(c * N + tail0, N - tail0)])

    return pl.kernel(body, out_type=out_type, mesh=mesh,
                     scratch_types=scratch)


def _make_sc_deg():
    RPT_D = ROWS_P // (2 * NS)
    IB_D = 8

    scratch = [
        pltpu.VMEM_SHARED((NA, H), jnp.float32),
        pltpu.VMEM((IB_D, NSUB, SUB), jnp.int32),
        pltpu.VMEM((SUB, H), jnp.float32),
    ]
    mesh = plsc.VectorSubcoreMesh(core_axis_name="c", subcore_axis_name="s")

    def body(dst_r, zacc, deg_out, dacc, idx_d, ones_v):
        c = lax.axis_index("c")
        s = lax.axis_index("s")
        nbase = s * ACC_PT
        tail0 = NS * ACC_PT
        tail_init = NA - tail0

        pltpu.sync_copy(zacc.at[pl.ds(nbase, ACC_PT)],
                        dacc.at[pl.ds(nbase, ACC_PT)])

        @pl.when(s == NS - 1)
        def _():
            pltpu.sync_copy(zacc.at[pl.ds(tail0, tail_init)],
                            dacc.at[pl.ds(tail0, tail_init)])

        def _fill(i, carry):
            for p in range(H // 16):
                ones_v[i, pl.ds(p * 16, 16)] = jnp.full((16,), 1.0,
                                                        jnp.float32)
            return carry
        lax.fori_loop(0, SUB, _fill, 0)

        lo = c * (ROWS_P // 2) + s * RPT_D
        plsc.subcore_barrier()

        def do_batch(jj, carry):
            j = carry
            for q in range(NSUB):
                pltpu.sync_copy(ones_v, dacc.at[idx_d.at[j, q]], add=True)
            return j + 1

        def do_block(b, carry):
            pltpu.sync_copy(dst_r.at[pl.ds(lo + b * IB_D, IB_D)], idx_d)
            lax.fori_loop(0, IB_D, do_batch, 0)
            return carry

        lax.fori_loop(0, RPT_D // IB_D, do_block, 0)
        plsc.subcore_barrier()

        pltpu.sync_copy(dacc.at[pl.ds(nbase, ACC_PT)],
                        deg_out.at[pl.ds(c * N + nbase, ACC_PT)])

        @pl.when(s == NS - 1)
        def _():
            pltpu.sync_copy(dacc.at[pl.ds(tail0, N - tail0)],
                            deg_out.at[pl.ds(c * N + tail0, N - tail0)])

    return pl.kernel(body,
                     out_type=[jax.ShapeDtypeStruct((2 * N, H), jnp.float32)],
                     mesh=mesh, scratch_types=scratch)


@functools.lru_cache(maxsize=None)
def _sc_agg(edge_wide):
    return _make_sc_agg(edge_wide=edge_wide)


@functools.lru_cache(maxsize=None)
def _sc_deg():
    return _make_sc_deg()


def _dense_body(agg_ref, deg_ref, x_ref, wn_ref, ws_ref, h_ref, sums_ref):
    i = pl.program_id(0)
    aggf = jnp.concatenate([agg_ref[0], agg_ref[1]], axis=1)
    xf = jnp.concatenate([x_ref[0], x_ref[1]], axis=1)
    d = deg_ref[0][:, 0:1] + deg_ref[1][:, 0:1]
    r = 1.0 / jnp.maximum(d, 1.0)
    h = (jnp.dot(aggf * r, wn_ref[...], preferred_element_type=jnp.float32)
         + jnp.dot(xf, ws_ref[...], preferred_element_type=jnp.float32))
    h_ref[...] = h

    @pl.when(i == 0)
    def _():
        sums_ref[...] = jnp.zeros_like(sums_ref)

    sums_ref[0:1, :] += jnp.sum(h, axis=0, keepdims=True)
    sums_ref[1:2, :] += jnp.sum(h * h, axis=0, keepdims=True)


def _dense(agg_st, deg, x_st, wn, ws):
    return pl.pallas_call(
        _dense_body,
        grid=(N // BN,),
        in_specs=[
            pl.BlockSpec((2, BN, H), lambda i: (0, i, 0)),
            pl.BlockSpec((2, BN, H), lambda i: (0, i, 0)),
            pl.BlockSpec((2, BN, H), lambda i: (0, i, 0)),
            pl.BlockSpec((D, D), lambda i: (0, 0)),
            pl.BlockSpec((D, D), lambda i: (0, 0)),
        ],
        out_specs=[
            pl.BlockSpec((BN, D), lambda i: (i, 0)),
            pl.BlockSpec((8, D), lambda i: (0, 0)),
        ],
        out_shape=[
            jax.ShapeDtypeStruct((N, D), jnp.float32),
            jax.ShapeDtypeStruct((8, D), jnp.float32),
        ],
    )(agg_st, deg, x_st, wn, ws)


def _norm_body(h_ref, sums_ref, g_ref, b_ref, out_ref, *, stacked):
    m = sums_ref[0:1, :] / float(N)
    v = sums_ref[1:2, :] / float(N) - m * m
    inv = lax.rsqrt(v + 1e-5)
    y = (h_ref[...] - m) * (inv * g_ref[...]) + b_ref[...]
    y = jnp.maximum(y, 0.0)
    if stacked:
        out_ref[0] = y[:, :H]
        out_ref[1] = y[:, H:]
    else:
        out_ref[...] = y


def _norm(h, sums, g, b, stacked):
    if stacked:
        out_spec = pl.BlockSpec((2, BN, H), lambda i: (0, i, 0))
        out_shape = jax.ShapeDtypeStruct((2, N, H), jnp.float32)
    else:
        out_spec = pl.BlockSpec((BN, D), lambda i: (i, 0))
        out_shape = jax.ShapeDtypeStruct((N, D), jnp.float32)
    return pl.pallas_call(
        functools.partial(_norm_body, stacked=stacked),
        grid=(N // BN,),
        in_specs=[
            pl.BlockSpec((BN, D), lambda i: (i, 0)),
            pl.BlockSpec((8, D), lambda i: (0, 0)),
            pl.BlockSpec((1, D), lambda i: (0, 0)),
            pl.BlockSpec((1, D), lambda i: (0, 0)),
        ],
        out_specs=out_spec,
        out_shape=out_shape,
    )(h, sums, g, b)


def _edge_body(e_ref, w_ref, b_ref, out_ref):
    y = jnp.dot(e_ref[...], w_ref[...], preferred_element_type=jnp.float32)
    y = jnp.maximum(y + b_ref[...], 0.0)
    out_ref[0] = y[:, :H]
    out_ref[1] = y[:, H:]


def _edge_tf(edge, w, b):
    return pl.pallas_call(
        _edge_body,
        grid=(E // BE,),
        in_specs=[
            pl.BlockSpec((BE, D), lambda i: (i, 0)),
            pl.BlockSpec((D, D), lambda i: (0, 0)),
            pl.BlockSpec((1, D), lambda i: (0, 0)),
        ],
        out_specs=pl.BlockSpec((2, BE, H), lambda i: (0, i, 0)),
        out_shape=jax.ShapeDtypeStruct((2, E, H), jnp.float32),
    )(edge, w, b)


def _rel_body(rel_ref, w1_ref, b1_ref, w2_ref, b2_ref, out_ref):
    y = jnp.dot(rel_ref[...], w1_ref[...], preferred_element_type=jnp.float32)
    y = jnp.maximum(y + b1_ref[...], 0.0)
    y = jnp.dot(y, w2_ref[...], preferred_element_type=jnp.float32)
    out_ref[...] = jnp.maximum(y + b2_ref[...], 0.0)


def _rel_mlp(rel, w1, b1, w2, b2):
    return pl.pallas_call(
        _rel_body,
        out_shape=jax.ShapeDtypeStruct(rel.shape, jnp.float32),
    )(rel, w1, b1, w2, b2)


def kernel(node_embs, edge_embs, rel_embs, edge_index,
           W_n1, W_s1, g1, b1, W_e1, be1,
           W_n2, W_s2, g2, b2, W_e2, be2):
    src = edge_index[0]
    dst = edge_index[1]

    node_st = jnp.stack([node_embs[:, :H], node_embs[:, H:]])
    npad = ROWS_P * CH - E
    src_pad = jnp.concatenate([src, jnp.zeros((npad,), jnp.int32)])
    src2_r = jnp.concatenate([src_pad, src_pad + N]).reshape(
        2 * ROWS_P, NSUB, SUB)
    dst_pad = jnp.concatenate(
        [dst, N + (jnp.arange(npad, dtype=jnp.int32) % NTRASH)])
    dst_r = dst_pad.reshape(ROWS_P, NSUB, SUB)
    zacc = jnp.zeros((NA, H), jnp.float32)

    (deg2,) = _sc_deg()(dst_r, zacc)
    deg = deg2.reshape(2, N, H)
    (agg1,) = _sc_agg(True)(node_st.reshape(2 * N, H), edge_embs,
                            src2_r, dst_r, zacc)
    h1, sums1 = _dense(agg1.reshape(2, N, H), deg, node_st, W_n1, W_s1)
    h1r_st = _norm(h1, sums1, g1.reshape(1, D), b1.reshape(1, D), stacked=True)
    e1r_st = _edge_tf(edge_embs, W_e1, be1.reshape(1, D))

    (agg2,) = _sc_agg(False)(h1r_st.reshape(2 * N, H),
                             e1r_st.reshape(2 * E, H),
                             src2_r, dst_r, zacc)
    h2, sums2 = _dense(agg2.reshape(2, N, H), deg, h1r_st, W_n2, W_s2)
    nodes_out = _norm(h2, sums2, g2.reshape(1, D), b2.reshape(1, D),
                      stacked=False)

    r = _rel_mlp(rel_embs, W_e1, be1.reshape(1, D), W_e2, be2.reshape(1, D))
    return (nodes_out, r)


# speedup vs baseline: 2.1924x; 1.1788x over previous
import functools

import jax
import jax.numpy as jnp
from jax import lax
from jax.experimental import pallas as pl
from jax.experimental.pallas import tpu as pltpu
from jax.experimental.pallas import tpu_sc as plsc

N = 10000
E = 160000
D = 256
H = 128
NS = 16
CH = 128
ROWS = E // CH
RPT = 80
ROWS_P = RPT * NS
NTRASH = 16
NA = N + NTRASH
ACC_PT = 624
IB = 16
SUB = 64
NSUB = CH // SUB
BN = 1000
BE = 1000


def _make_sc_agg(edge_wide):
    out_type = [jax.ShapeDtypeStruct((2 * N, H), jnp.float32)]

    scratch = [
        pltpu.VMEM_SHARED((NA, H), jnp.float32),
        pltpu.VMEM((IB, NSUB, SUB), jnp.int32),
        pltpu.VMEM((IB, NSUB, SUB), jnp.int32),
        pltpu.VMEM((SUB, H), jnp.float32),
        pltpu.VMEM((SUB, H), jnp.float32),
        pltpu.VMEM((SUB, H), jnp.float32),
        pltpu.VMEM((SUB, H), jnp.float32),
        pltpu.SemaphoreType.DMA,
        pltpu.SemaphoreType.DMA,
        pltpu.SemaphoreType.DMA,
        pltpu.SemaphoreType.DMA,
    ]

    mesh = plsc.VectorSubcoreMesh(core_axis_name="c", subcore_axis_name="s")

    def body(node_t, edge_t, src2_r, dst_r, zacc, agg_out,
             acc, idx_s, idx_d, gath_a, gath_b, edge_a, edge_b,
             sem_a, sem_b, ssem_a, ssem_b):
        c = lax.axis_index("c")
        s = lax.axis_index("s")
        nbase = s * ACC_PT
        tail0 = NS * ACC_PT
        tail_init = NA - tail0

        pltpu.sync_copy(zacc.at[pl.ds(nbase, ACC_PT)],
                        acc.at[pl.ds(nbase, ACC_PT)])

        @pl.when(s == NS - 1)
        def _():
            pltpu.sync_copy(zacc.at[pl.ds(tail0, tail_init)],
                            acc.at[pl.ds(tail0, tail_init)])

        lo = s * RPT

        def edge_src(row, q):
            e0 = row * CH + q * SUB
            if edge_wide:
                return edge_t.at[pl.ds(e0, SUB), pl.ds(c * H, H)]
            return edge_t.at[pl.ds(c * E + e0, SUB)]

        def issue_in(j, q, gbuf, ebuf, sem, blk0):
            row = jnp.minimum(blk0 + j, ROWS - 1)
            pltpu.async_copy(node_t.at[idx_s.at[j, q]], gbuf, sem)
            pltpu.async_copy(edge_src(row, q), ebuf, sem)

        def wait_in(j, q, gbuf, ebuf, sem, blk0):
            row = jnp.minimum(blk0 + j, ROWS - 1)
            pltpu.make_async_copy(node_t.at[idx_s.at[j, q]], gbuf, sem).wait()
            pltpu.make_async_copy(edge_src(row, q), ebuf, sem).wait()

        def mul(gbuf, ebuf):
            def mulrow(r, carry2):
                for p in range(H // 16):
                    sl = pl.ds(p * 16, 16)
                    gbuf[r, sl] = gbuf[r, sl] * ebuf[r, sl]
                return carry2
            lax.fori_loop(0, SUB, mulrow, 0, unroll=4)

        def scat(j, q, gbuf, sem):
            pltpu.async_copy(gbuf, acc.at[idx_d.at[j, q]], sem, add=True)

        def scat_wait(j, q, gbuf, sem):
            pltpu.make_async_copy(gbuf, acc.at[idx_d.at[j, q]], sem).wait()

        plsc.subcore_barrier()

        for blk in range(RPT // IB):
            blk0 = lo + blk * IB
            pltpu.sync_copy(src2_r.at[pl.ds(c * ROWS_P + blk0, IB)], idx_s)
            pltpu.sync_copy(dst_r.at[pl.ds(blk0, IB)], idx_d)
            issue_in(0, 0, gath_a, edge_a, sem_a, blk0)

            def pair(j, carry):
                wait_in(j, 0, gath_a, edge_a, sem_a, blk0)

                @pl.when(j > 0)
                def _():
                    scat_wait(j - 1, 1, gath_b, ssem_b)
                issue_in(j, 1, gath_b, edge_b, sem_b, blk0)
                mul(gath_a, edge_a)
                scat(j, 0, gath_a, ssem_a)
                wait_in(j, 1, gath_b, edge_b, sem_b, blk0)
                scat_wait(j, 0, gath_a, ssem_a)

                @pl.when(j < IB - 1)
                def _():
                    issue_in(j + 1, 0, gath_a, edge_a, sem_a, blk0)
                mul(gath_b, edge_b)
                scat(j, 1, gath_b, ssem_b)
                return carry

            lax.fori_loop(0, IB, pair, 0)
            scat_wait(IB - 1, 1, gath_b, ssem_b)

        plsc.subcore_barrier()

        pltpu.sync_copy(acc.at[pl.ds(nbase, ACC_PT)],
                        agg_out.at[pl.ds(c * N + nbase, ACC_PT)])

        @pl.when(s == NS - 1)
        def _():
            pltpu.sync_copy(acc.at[pl.ds(tail0, N - tail0)],
                            agg_out.at[pl.ds(c * N + tail0, N - tail0)])

    return pl.kernel(body, out_type=out_type, mesh=mesh,
                     scratch_types=scratch)


def _make_sc_deg():
    RPT_D = ROWS_P // (2 * NS)
    IB_D = 8

    scratch = [
        pltpu.VMEM_SHARED((NA, H), jnp.float32),
        pltpu.VMEM((IB_D, NSUB, SUB), jnp.int32),
        pltpu.VMEM((SUB, H), jnp.float32),
    ]
    mesh = plsc.VectorSubcoreMesh(core_axis_name="c", subcore_axis_name="s")

    def body(dst_r, zacc, deg_out, dacc, idx_d, ones_v):
        c = lax.axis_index("c")
        s = lax.axis_index("s")
        nbase = s * ACC_PT
        tail0 = NS * ACC_PT
        tail_init = NA - tail0

        pltpu.sync_copy(zacc.at[pl.ds(nbase, ACC_PT)],
                        dacc.at[pl.ds(nbase, ACC_PT)])

        @pl.when(s == NS - 1)
        def _():
            pltpu.sync_copy(zacc.at[pl.ds(tail0, tail_init)],
                            dacc.at[pl.ds(tail0, tail_init)])

        def _fill(i, carry):
            for p in range(H // 16):
                ones_v[i, pl.ds(p * 16, 16)] = jnp.full((16,), 1.0,
                                                        jnp.float32)
            return carry
        lax.fori_loop(0, SUB, _fill, 0)

        lo = c * (ROWS_P // 2) + s * RPT_D
        plsc.subcore_barrier()

        def do_batch(jj, carry):
            j = carry
            for q in range(NSUB):
                pltpu.sync_copy(ones_v, dacc.at[idx_d.at[j, q]], add=True)
            return j + 1

        def do_block(b, carry):
            pltpu.sync_copy(dst_r.at[pl.ds(lo + b * IB_D, IB_D)], idx_d)
            lax.fori_loop(0, IB_D, do_batch, 0)
            return carry

        lax.fori_loop(0, RPT_D // IB_D, do_block, 0)
        plsc.subcore_barrier()

        pltpu.sync_copy(dacc.at[pl.ds(nbase, ACC_PT)],
                        deg_out.at[pl.ds(c * N + nbase, ACC_PT)])

        @pl.when(s == NS - 1)
        def _():
            pltpu.sync_copy(dacc.at[pl.ds(tail0, N - tail0)],
                            deg_out.at[pl.ds(c * N + tail0, N - tail0)])

    return pl.kernel(body,
                     out_type=[jax.ShapeDtypeStruct((2 * N, H), jnp.float32)],
                     mesh=mesh, scratch_types=scratch)


@functools.lru_cache(maxsize=None)
def _sc_agg(edge_wide):
    return _make_sc_agg(edge_wide=edge_wide)


@functools.lru_cache(maxsize=None)
def _sc_deg():
    return _make_sc_deg()


def _dense_body(agg_ref, deg_ref, x_ref, wn_ref, ws_ref, h_ref, sums_ref):
    i = pl.program_id(0)
    aggf = jnp.concatenate([agg_ref[0], agg_ref[1]], axis=1)
    xf = jnp.concatenate([x_ref[0], x_ref[1]], axis=1)
    d = deg_ref[0][:, 0:1] + deg_ref[1][:, 0:1]
    r = 1.0 / jnp.maximum(d, 1.0)
    h = (jnp.dot(aggf * r, wn_ref[...], preferred_element_type=jnp.float32)
         + jnp.dot(xf, ws_ref[...], preferred_element_type=jnp.float32))
    h_ref[...] = h

    @pl.when(i == 0)
    def _():
        sums_ref[...] = jnp.zeros_like(sums_ref)

    sums_ref[0:1, :] += jnp.sum(h, axis=0, keepdims=True)
    sums_ref[1:2, :] += jnp.sum(h * h, axis=0, keepdims=True)


def _dense(agg_st, deg, x_st, wn, ws):
    return pl.pallas_call(
        _dense_body,
        grid=(N // BN,),
        in_specs=[
            pl.BlockSpec((2, BN, H), lambda i: (0, i, 0)),
            pl.BlockSpec((2, BN, H), lambda i: (0, i, 0)),
            pl.BlockSpec((2, BN, H), lambda i: (0, i, 0)),
            pl.BlockSpec((D, D), lambda i: (0, 0)),
            pl.BlockSpec((D, D), lambda i: (0, 0)),
        ],
        out_specs=[
            pl.BlockSpec((BN, D), lambda i: (i, 0)),
            pl.BlockSpec((8, D), lambda i: (0, 0)),
        ],
        out_shape=[
            jax.ShapeDtypeStruct((N, D), jnp.float32),
            jax.ShapeDtypeStruct((8, D), jnp.float32),
        ],
    )(agg_st, deg, x_st, wn, ws)


def _norm_body(h_ref, sums_ref, g_ref, b_ref, out_ref, *, stacked):
    m = sums_ref[0:1, :] / float(N)
    v = sums_ref[1:2, :] / float(N) - m * m
    inv = lax.rsqrt(v + 1e-5)
    y = (h_ref[...] - m) * (inv * g_ref[...]) + b_ref[...]
    y = jnp.maximum(y, 0.0)
    if stacked:
        out_ref[0] = y[:, :H]
        out_ref[1] = y[:, H:]
    else:
        out_ref[...] = y


def _norm(h, sums, g, b, stacked):
    if stacked:
        out_spec = pl.BlockSpec((2, BN, H), lambda i: (0, i, 0))
        out_shape = jax.ShapeDtypeStruct((2, N, H), jnp.float32)
    else:
        out_spec = pl.BlockSpec((BN, D), lambda i: (i, 0))
        out_shape = jax.ShapeDtypeStruct((N, D), jnp.float32)
    return pl.pallas_call(
        functools.partial(_norm_body, stacked=stacked),
        grid=(N // BN,),
        in_specs=[
            pl.BlockSpec((BN, D), lambda i: (i, 0)),
            pl.BlockSpec((8, D), lambda i: (0, 0)),
            pl.BlockSpec((1, D), lambda i: (0, 0)),
            pl.BlockSpec((1, D), lambda i: (0, 0)),
        ],
        out_specs=out_spec,
        out_shape=out_shape,
    )(h, sums, g, b)


def _edge_body(e_ref, w_ref, b_ref, out_ref):
    y = jnp.dot(e_ref[...], w_ref[...], preferred_element_type=jnp.float32)
    y = jnp.maximum(y + b_ref[...], 0.0)
    out_ref[0] = y[:, :H]
    out_ref[1] = y[:, H:]


def _edge_tf(edge, w, b):
    return pl.pallas_call(
        _edge_body,
        grid=(E // BE,),
        in_specs=[
            pl.BlockSpec((BE, D), lambda i: (i, 0)),
            pl.BlockSpec((D, D), lambda i: (0, 0)),
            pl.BlockSpec((1, D), lambda i: (0, 0)),
        ],
        out_specs=pl.BlockSpec((2, BE, H), lambda i: (0, i, 0)),
        out_shape=jax.ShapeDtypeStruct((2, E, H), jnp.float32),
    )(edge, w, b)


def _rel_body(rel_ref, w1_ref, b1_ref, w2_ref, b2_ref, out_ref):
    y = jnp.dot(rel_ref[...], w1_ref[...], preferred_element_type=jnp.float32)
    y = jnp.maximum(y + b1_ref[...], 0.0)
    y = jnp.dot(y, w2_ref[...], preferred_element_type=jnp.float32)
    out_ref[...] = jnp.maximum(y + b2_ref[...], 0.0)


def _rel_mlp(rel, w1, b1, w2, b2):
    return pl.pallas_call(
        _rel_body,
        out_shape=jax.ShapeDtypeStruct(rel.shape, jnp.float32),
    )(rel, w1, b1, w2, b2)


def kernel(node_embs, edge_embs, rel_embs, edge_index,
           W_n1, W_s1, g1, b1, W_e1, be1,
           W_n2, W_s2, g2, b2, W_e2, be2):
    src = edge_index[0]
    dst = edge_index[1]

    node_st = jnp.stack([node_embs[:, :H], node_embs[:, H:]])
    npad = ROWS_P * CH - E
    src_pad = jnp.concatenate([src, jnp.zeros((npad,), jnp.int32)])
    src2_r = jnp.concatenate([src_pad, src_pad + N]).reshape(
        2 * ROWS_P, NSUB, SUB)
    dst_pad = jnp.concatenate(
        [dst, N + (jnp.arange(npad, dtype=jnp.int32) % NTRASH)])
    dst_r = dst_pad.reshape(ROWS_P, NSUB, SUB)
    zacc = jnp.zeros((NA, H), jnp.float32)

    (deg2,) = _sc_deg()(dst_r, zacc)
    deg = deg2.reshape(2, N, H)
    (agg1,) = _sc_agg(True)(node_st.reshape(2 * N, H), edge_embs,
                            src2_r, dst_r, zacc)
    h1, sums1 = _dense(agg1.reshape(2, N, H), deg, node_st, W_n1, W_s1)
    h1r_st = _norm(h1, sums1, g1.reshape(1, D), b1.reshape(1, D), stacked=True)
    e1r_st = _edge_tf(edge_embs, W_e1, be1.reshape(1, D))

    (agg2,) = _sc_agg(False)(h1r_st.reshape(2 * N, H),
                             e1r_st.reshape(2 * E, H),
                             src2_r, dst_r, zacc)
    h2, sums2 = _dense(agg2.reshape(2, N, H), deg, h1r_st, W_n2, W_s2)
    nodes_out = _norm(h2, sums2, g2.reshape(1, D), b2.reshape(1, D),
                      stacked=False)

    r = _rel_mlp(rel_embs, W_e1, be1.reshape(1, D), W_e2, be2.reshape(1, D))
    return (nodes_out, r)


# speedup vs baseline: 2.3413x; 1.0679x over previous
import functools

import jax
import jax.numpy as jnp
from jax import lax
from jax.experimental import pallas as pl
from jax.experimental.pallas import tpu as pltpu
from jax.experimental.pallas import tpu_sc as plsc

N = 10000
E = 160000
D = 256
H = 128
NS = 16
CH = 128
ROWS = E // CH
RPT = 80
ROWS_P = RPT * NS
NTRASH = 16
NA = N + NTRASH
ACC_PT = 624
IB = 16
SUB = 32
NSUB = CH // SUB
BN = 1000
BE = 1000


def _make_sc_agg(edge_wide):
    out_type = [jax.ShapeDtypeStruct((2 * N, H), jnp.float32)]

    scratch = [
        pltpu.VMEM_SHARED((NA, H), jnp.float32),
        pltpu.VMEM((IB, NSUB, SUB), jnp.int32),
        pltpu.VMEM((IB, NSUB, SUB), jnp.int32),
    ] + [pltpu.VMEM((SUB, H), jnp.float32) for _ in range(NSUB)] \
      + [pltpu.VMEM((SUB, H), jnp.float32) for _ in range(NSUB)] \
      + [pltpu.SemaphoreType.DMA for _ in range(2 * NSUB)]

    mesh = plsc.VectorSubcoreMesh(core_axis_name="c", subcore_axis_name="s")

    def body(node_t, edge_t, src2_r, dst_r, zacc, agg_out,
             acc, idx_s, idx_d, *bufs):
        ga = bufs[:NSUB]
        ea = bufs[NSUB:2 * NSUB]
        isem = bufs[2 * NSUB:3 * NSUB]
        osem = bufs[3 * NSUB:4 * NSUB]
        c = lax.axis_index("c")
        s = lax.axis_index("s")
        nbase = s * ACC_PT
        tail0 = NS * ACC_PT
        tail_init = NA - tail0

        pltpu.sync_copy(zacc.at[pl.ds(nbase, ACC_PT)],
                        acc.at[pl.ds(nbase, ACC_PT)])

        @pl.when(s == NS - 1)
        def _():
            pltpu.sync_copy(zacc.at[pl.ds(tail0, tail_init)],
                            acc.at[pl.ds(tail0, tail_init)])

        lo = s * RPT

        def edge_src(row, q):
            e0 = row * CH + q * SUB
            if edge_wide:
                return edge_t.at[pl.ds(e0, SUB), pl.ds(c * H, H)]
            return edge_t.at[pl.ds(c * E + e0, SUB)]

        def issue_in(j, q, gbuf, ebuf, sem, blk0):
            row = jnp.minimum(blk0 + j, ROWS - 1)
            pltpu.async_copy(node_t.at[idx_s.at[j, q]], gbuf, sem)
            pltpu.async_copy(edge_src(row, q), ebuf, sem)

        def wait_in(j, q, gbuf, ebuf, sem, blk0):
            row = jnp.minimum(blk0 + j, ROWS - 1)
            pltpu.make_async_copy(node_t.at[idx_s.at[j, q]], gbuf, sem).wait()
            pltpu.make_async_copy(edge_src(row, q), ebuf, sem).wait()

        def mul(gbuf, ebuf):
            def mulrow(r, carry2):
                for p in range(H // 16):
                    sl = pl.ds(p * 16, 16)
                    gbuf[r, sl] = gbuf[r, sl] * ebuf[r, sl]
                return carry2
            lax.fori_loop(0, SUB, mulrow, 0, unroll=4)

        def scat(j, q, gbuf, sem):
            pltpu.async_copy(gbuf, acc.at[idx_d.at[j, q]], sem, add=True)

        def scat_wait(j, q, gbuf, sem):
            pltpu.make_async_copy(gbuf, acc.at[idx_d.at[j, q]], sem).wait()

        plsc.subcore_barrier()

        for blk in range(RPT // IB):
            blk0 = lo + blk * IB
            pltpu.sync_copy(src2_r.at[pl.ds(c * ROWS_P + blk0, IB)], idx_s)
            pltpu.sync_copy(dst_r.at[pl.ds(blk0, IB)], idx_d)
            issue_in(0, 0, ga[0], ea[0], isem[0], blk0)
            issue_in(0, 1, ga[1], ea[1], isem[1], blk0)

            def row(j, carry):
                for q in range(NSUB):
                    wait_in(j, q, ga[q], ea[q], isem[q], blk0)
                    mul(ga[q], ea[q])
                    q2 = (q + 2) % NSUB
                    if q < 2:
                        @pl.when(j > 0)
                        def _():
                            scat_wait(j - 1, q2, ga[q2], osem[q2])
                        issue_in(j, q + 2, ga[q2], ea[q2], isem[q2], blk0)
                    else:
                        scat_wait(j, q2, ga[q2], osem[q2])

                        @pl.when(j < IB - 1)
                        def _():
                            issue_in(j + 1, q2, ga[q2], ea[q2], isem[q2],
                                     blk0)
                    scat(j, q, ga[q], osem[q])
                return carry

            lax.fori_loop(0, IB, row, 0)
            scat_wait(IB - 1, 2, ga[2], osem[2])
            scat_wait(IB - 1, 3, ga[3], osem[3])

        plsc.subcore_barrier()

        pltpu.sync_copy(acc.at[pl.ds(nbase, ACC_PT)],
                        agg_out.at[pl.ds(c * N + nbase, ACC_PT)])

        @pl.when(s == NS - 1)
        def _():
            pltpu.sync_copy(acc.at[pl.ds(tail0, N - tail0)],
                            agg_out.at[pl.ds(c * N + tail0, N - tail0)])

    return pl.kernel(body, out_type=out_type, mesh=mesh,
                     scratch_types=scratch)


def _make_sc_deg():
    RPT_D = ROWS_P // (2 * NS)
    IB_D = 8

    scratch = [
        pltpu.VMEM_SHARED((NA, H), jnp.float32),
        pltpu.VMEM((IB_D, NSUB, SUB), jnp.int32),
        pltpu.VMEM((SUB, H), jnp.float32),
    ]
    mesh = plsc.VectorSubcoreMesh(core_axis_name="c", subcore_axis_name="s")

    def body(dst_r, zacc, deg_out, dacc, idx_d, ones_v):
        c = lax.axis_index("c")
        s = lax.axis_index("s")
        nbase = s * ACC_PT
        tail0 = NS * ACC_PT
        tail_init = NA - tail0

        pltpu.sync_copy(zacc.at[pl.ds(nbase, ACC_PT)],
                        dacc.at[pl.ds(nbase, ACC_PT)])

        @pl.when(s == NS - 1)
        def _():
            pltpu.sync_copy(zacc.at[pl.ds(tail0, tail_init)],
                            dacc.at[pl.ds(tail0, tail_init)])

        def _fill(i, carry):
            for p in range(H // 16):
                ones_v[i, pl.ds(p * 16, 16)] = jnp.full((16,), 1.0,
                                                        jnp.float32)
            return carry
        lax.fori_loop(0, SUB, _fill, 0)

        lo = c * (ROWS_P // 2) + s * RPT_D
        plsc.subcore_barrier()

        def do_batch(jj, carry):
            j = carry
            for q in range(NSUB):
                pltpu.sync_copy(ones_v, dacc.at[idx_d.at[j, q]], add=True)
            return j + 1

        def do_block(b, carry):
            pltpu.sync_copy(dst_r.at[pl.ds(lo + b * IB_D, IB_D)], idx_d)
            lax.fori_loop(0, IB_D, do_batch, 0)
            return carry

        lax.fori_loop(0, RPT_D // IB_D, do_block, 0)
        plsc.subcore_barrier()

        pltpu.sync_copy(dacc.at[pl.ds(nbase, ACC_PT)],
                        deg_out.at[pl.ds(c * N + nbase, ACC_PT)])

        @pl.when(s == NS - 1)
        def _():
            pltpu.sync_copy(dacc.at[pl.ds(tail0, N - tail0)],
                            deg_out.at[pl.ds(c * N + tail0, N - tail0)])

    return pl.kernel(body,
                     out_type=[jax.ShapeDtypeStruct((2 * N, H), jnp.float32)],
                     mesh=mesh, scratch_types=scratch)


@functools.lru_cache(maxsize=None)
def _sc_agg(edge_wide):
    return _make_sc_agg(edge_wide=edge_wide)


@functools.lru_cache(maxsize=None)
def _sc_deg():
    return _make_sc_deg()


def _dense_body(agg_ref, deg_ref, x_ref, wn_ref, ws_ref, h_ref, sums_ref):
    i = pl.program_id(0)
    aggf = jnp.concatenate([agg_ref[0], agg_ref[1]], axis=1)
    xf = jnp.concatenate([x_ref[0], x_ref[1]], axis=1)
    d = deg_ref[0][:, 0:1] + deg_ref[1][:, 0:1]
    r = 1.0 / jnp.maximum(d, 1.0)
    h = (jnp.dot(aggf * r, wn_ref[...], preferred_element_type=jnp.float32)
         + jnp.dot(xf, ws_ref[...], preferred_element_type=jnp.float32))
    h_ref[...] = h

    @pl.when(i == 0)
    def _():
        sums_ref[...] = jnp.zeros_like(sums_ref)

    sums_ref[0:1, :] += jnp.sum(h, axis=0, keepdims=True)
    sums_ref[1:2, :] += jnp.sum(h * h, axis=0, keepdims=True)


def _dense(agg_st, deg, x_st, wn, ws):
    return pl.pallas_call(
        _dense_body,
        grid=(N // BN,),
        in_specs=[
            pl.BlockSpec((2, BN, H), lambda i: (0, i, 0)),
            pl.BlockSpec((2, BN, H), lambda i: (0, i, 0)),
            pl.BlockSpec((2, BN, H), lambda i: (0, i, 0)),
            pl.BlockSpec((D, D), lambda i: (0, 0)),
            pl.BlockSpec((D, D), lambda i: (0, 0)),
        ],
        out_specs=[
            pl.BlockSpec((BN, D), lambda i: (i, 0)),
            pl.BlockSpec((8, D), lambda i: (0, 0)),
        ],
        out_shape=[
            jax.ShapeDtypeStruct((N, D), jnp.float32),
            jax.ShapeDtypeStruct((8, D), jnp.float32),
        ],
    )(agg_st, deg, x_st, wn, ws)


def _norm_body(h_ref, sums_ref, g_ref, b_ref, out_ref, *, stacked):
    m = sums_ref[0:1, :] / float(N)
    v = sums_ref[1:2, :] / float(N) - m * m
    inv = lax.rsqrt(v + 1e-5)
    y = (h_ref[...] - m) * (inv * g_ref[...]) + b_ref[...]
    y = jnp.maximum(y, 0.0)
    if stacked:
        out_ref[0] = y[:, :H]
        out_ref[1] = y[:, H:]
    else:
        out_ref[...] = y


def _norm(h, sums, g, b, stacked):
    if stacked:
        out_spec = pl.BlockSpec((2, BN, H), lambda i: (0, i, 0))
        out_shape = jax.ShapeDtypeStruct((2, N, H), jnp.float32)
    else:
        out_spec = pl.BlockSpec((BN, D), lambda i: (i, 0))
        out_shape = jax.ShapeDtypeStruct((N, D), jnp.float32)
    return pl.pallas_call(
        functools.partial(_norm_body, stacked=stacked),
        grid=(N // BN,),
        in_specs=[
            pl.BlockSpec((BN, D), lambda i: (i, 0)),
            pl.BlockSpec((8, D), lambda i: (0, 0)),
            pl.BlockSpec((1, D), lambda i: (0, 0)),
            pl.BlockSpec((1, D), lambda i: (0, 0)),
        ],
        out_specs=out_spec,
        out_shape=out_shape,
    )(h, sums, g, b)


def _edge_body(e_ref, w_ref, b_ref, out_ref):
    y = jnp.dot(e_ref[...], w_ref[...], preferred_element_type=jnp.float32)
    y = jnp.maximum(y + b_ref[...], 0.0)
    out_ref[0] = y[:, :H]
    out_ref[1] = y[:, H:]


def _edge_tf(edge, w, b):
    return pl.pallas_call(
        _edge_body,
        grid=(E // BE,),
        in_specs=[
            pl.BlockSpec((BE, D), lambda i: (i, 0)),
            pl.BlockSpec((D, D), lambda i: (0, 0)),
            pl.BlockSpec((1, D), lambda i: (0, 0)),
        ],
        out_specs=pl.BlockSpec((2, BE, H), lambda i: (0, i, 0)),
        out_shape=jax.ShapeDtypeStruct((2, E, H), jnp.float32),
    )(edge, w, b)


def _rel_body(rel_ref, w1_ref, b1_ref, w2_ref, b2_ref, out_ref):
    y = jnp.dot(rel_ref[...], w1_ref[...], preferred_element_type=jnp.float32)
    y = jnp.maximum(y + b1_ref[...], 0.0)
    y = jnp.dot(y, w2_ref[...], preferred_element_type=jnp.float32)
    out_ref[...] = jnp.maximum(y + b2_ref[...], 0.0)


def _rel_mlp(rel, w1, b1, w2, b2):
    return pl.pallas_call(
        _rel_body,
        out_shape=jax.ShapeDtypeStruct(rel.shape, jnp.float32),
    )(rel, w1, b1, w2, b2)


def kernel(node_embs, edge_embs, rel_embs, edge_index,
           W_n1, W_s1, g1, b1, W_e1, be1,
           W_n2, W_s2, g2, b2, W_e2, be2):
    src = edge_index[0]
    dst = edge_index[1]

    node_st = jnp.stack([node_embs[:, :H], node_embs[:, H:]])
    npad = ROWS_P * CH - E
    src_pad = jnp.concatenate([src, jnp.zeros((npad,), jnp.int32)])
    src2_r = jnp.concatenate([src_pad, src_pad + N]).reshape(
        2 * ROWS_P, NSUB, SUB)
    dst_pad = jnp.concatenate(
        [dst, N + (jnp.arange(npad, dtype=jnp.int32) % NTRASH)])
    dst_r = dst_pad.reshape(ROWS_P, NSUB, SUB)
    zacc = jnp.zeros((NA, H), jnp.float32)

    (deg2,) = _sc_deg()(dst_r, zacc)
    deg = deg2.reshape(2, N, H)
    (agg1,) = _sc_agg(True)(node_st.reshape(2 * N, H), edge_embs,
                            src2_r, dst_r, zacc)
    h1, sums1 = _dense(agg1.reshape(2, N, H), deg, node_st, W_n1, W_s1)
    h1r_st = _norm(h1, sums1, g1.reshape(1, D), b1.reshape(1, D), stacked=True)
    e1r_st = _edge_tf(edge_embs, W_e1, be1.reshape(1, D))

    (agg2,) = _sc_agg(False)(h1r_st.reshape(2 * N, H),
                             e1r_st.reshape(2 * E, H),
                             src2_r, dst_r, zacc)
    h2, sums2 = _dense(agg2.reshape(2, N, H), deg, h1r_st, W_n2, W_s2)
    nodes_out = _norm(h2, sums2, g2.reshape(1, D), b2.reshape(1, D),
                      stacked=False)

    r = _rel_mlp(rel_embs, W_e1, be1.reshape(1, D), W_e2, be2.reshape(1, D))
    return (nodes_out, r)


# speedup vs baseline: 2.3421x; 1.0003x over previous
import functools

import jax
import jax.numpy as jnp
from jax import lax
from jax.experimental import pallas as pl
from jax.experimental.pallas import tpu as pltpu
from jax.experimental.pallas import tpu_sc as plsc

N = 10000
E = 160000
D = 256
H = 128
NS = 16
CH = 128
ROWS = E // CH
RPT = 80
ROWS_P = RPT * NS
NTRASH = 16
NA = N + NTRASH
ACC_PT = 624
IB = 16
SUB = 32
NSUB = CH // SUB
BN = 1000
BE = 1000


def _make_sc_agg(edge_wide):
    out_type = [jax.ShapeDtypeStruct((2 * N, H), jnp.float32)]

    scratch = [
        pltpu.VMEM_SHARED((NA, H), jnp.float32),
        pltpu.VMEM((IB, NSUB, SUB), jnp.int32),
        pltpu.VMEM((IB, NSUB, SUB), jnp.int32),
    ] + [pltpu.VMEM((SUB, H), jnp.float32) for _ in range(NSUB)] \
      + [pltpu.VMEM((SUB, H), jnp.float32) for _ in range(NSUB)] \
      + [pltpu.SemaphoreType.DMA for _ in range(2 * NSUB)]

    mesh = plsc.VectorSubcoreMesh(core_axis_name="c", subcore_axis_name="s")

    def body(node_t, edge_t, src2_r, dst_r, zacc, agg_out,
             acc, idx_s, idx_d, *bufs):
        ga = bufs[:NSUB]
        ea = bufs[NSUB:2 * NSUB]
        isem = bufs[2 * NSUB:3 * NSUB]
        osem = bufs[3 * NSUB:4 * NSUB]
        c = lax.axis_index("c")
        s = lax.axis_index("s")
        nbase = s * ACC_PT
        tail0 = NS * ACC_PT
        tail_init = NA - tail0

        pltpu.sync_copy(zacc.at[pl.ds(nbase, ACC_PT)],
                        acc.at[pl.ds(nbase, ACC_PT)])

        @pl.when(s == NS - 1)
        def _():
            pltpu.sync_copy(zacc.at[pl.ds(tail0, tail_init)],
                            acc.at[pl.ds(tail0, tail_init)])

        lo = s * RPT

        def edge_src(row, q):
            e0 = row * CH + q * SUB
            if edge_wide:
                return edge_t.at[pl.ds(e0, SUB), pl.ds(c * H, H)]
            return edge_t.at[pl.ds(c * E + e0, SUB)]

        def issue_in(j, q, gbuf, ebuf, sem, blk0):
            row = jnp.minimum(blk0 + j, ROWS - 1)
            pltpu.async_copy(node_t.at[idx_s.at[j, q]], gbuf, sem)
            pltpu.async_copy(edge_src(row, q), ebuf, sem)

        def wait_in(j, q, gbuf, ebuf, sem, blk0):
            row = jnp.minimum(blk0 + j, ROWS - 1)
            pltpu.make_async_copy(node_t.at[idx_s.at[j, q]], gbuf, sem).wait()
            pltpu.make_async_copy(edge_src(row, q), ebuf, sem).wait()

        def mul(gbuf, ebuf):
            def mulrow(r, carry2):
                for p in range(H // 16):
                    sl = pl.ds(p * 16, 16)
                    gbuf[r, sl] = gbuf[r, sl] * ebuf[r, sl]
                return carry2
            lax.fori_loop(0, SUB, mulrow, 0, unroll=4)

        def scat(j, q, gbuf, sem):
            pltpu.async_copy(gbuf, acc.at[idx_d.at[j, q]], sem, add=True)

        def scat_wait(j, q, gbuf, sem):
            pltpu.make_async_copy(gbuf, acc.at[idx_d.at[j, q]], sem).wait()

        plsc.subcore_barrier()

        for blk in range(RPT // IB):
            blk0 = lo + blk * IB
            pltpu.sync_copy(src2_r.at[pl.ds(c * ROWS_P + blk0, IB)], idx_s)
            pltpu.sync_copy(dst_r.at[pl.ds(blk0, IB)], idx_d)
            issue_in(0, 0, ga[0], ea[0], isem[0], blk0)
            issue_in(0, 1, ga[1], ea[1], isem[1], blk0)

            def row(j, carry):
                for q in range(NSUB):
                    wait_in(j, q, ga[q], ea[q], isem[q], blk0)
                    mul(ga[q], ea[q])
                    q2 = (q + 2) % NSUB
                    if q < 2:
                        @pl.when(j > 0)
                        def _():
                            scat_wait(j - 1, q2, ga[q2], osem[q2])
                        issue_in(j, q + 2, ga[q2], ea[q2], isem[q2], blk0)
                    else:
                        scat_wait(j, q2, ga[q2], osem[q2])

                        @pl.when(j < IB - 1)
                        def _():
                            issue_in(j + 1, q2, ga[q2], ea[q2], isem[q2],
                                     blk0)
                    scat(j, q, ga[q], osem[q])
                return carry

            lax.fori_loop(0, IB, row, 0)
            scat_wait(IB - 1, 2, ga[2], osem[2])
            scat_wait(IB - 1, 3, ga[3], osem[3])

        plsc.subcore_barrier()

        pltpu.sync_copy(acc.at[pl.ds(nbase, ACC_PT)],
                        agg_out.at[pl.ds(c * N + nbase, ACC_PT)])

        @pl.when(s == NS - 1)
        def _():
            pltpu.sync_copy(acc.at[pl.ds(tail0, N - tail0)],
                            agg_out.at[pl.ds(c * N + tail0, N - tail0)])

    return pl.kernel(body, out_type=out_type, mesh=mesh,
                     scratch_types=scratch)


def _make_sc_deg():
    RPT_D = ROWS_P // (2 * NS)
    IB_D = 8

    scratch = [
        pltpu.VMEM_SHARED((NA, H), jnp.float32),
        pltpu.VMEM((IB_D, NSUB, SUB), jnp.int32),
        pltpu.VMEM((SUB, H), jnp.float32),
    ]
    mesh = plsc.VectorSubcoreMesh(core_axis_name="c", subcore_axis_name="s")

    def body(dst_r, zacc, deg_out, dacc, idx_d, ones_v):
        c = lax.axis_index("c")
        s = lax.axis_index("s")
        nbase = s * ACC_PT
        tail0 = NS * ACC_PT
        tail_init = NA - tail0

        pltpu.sync_copy(zacc.at[pl.ds(nbase, ACC_PT)],
                        dacc.at[pl.ds(nbase, ACC_PT)])

        @pl.when(s == NS - 1)
        def _():
            pltpu.sync_copy(zacc.at[pl.ds(tail0, tail_init)],
                            dacc.at[pl.ds(tail0, tail_init)])

        def _fill(i, carry):
            for p in range(H // 16):
                ones_v[i, pl.ds(p * 16, 16)] = jnp.full((16,), 1.0,
                                                        jnp.float32)
            return carry
        lax.fori_loop(0, SUB, _fill, 0)

        lo = c * (ROWS_P // 2) + s * RPT_D
        plsc.subcore_barrier()

        def do_batch(jj, carry):
            j = carry
            for q in range(NSUB):
                pltpu.sync_copy(ones_v, dacc.at[idx_d.at[j, q]], add=True)
            return j + 1

        def do_block(b, carry):
            pltpu.sync_copy(dst_r.at[pl.ds(lo + b * IB_D, IB_D)], idx_d)
            lax.fori_loop(0, IB_D, do_batch, 0)
            return carry

        lax.fori_loop(0, RPT_D // IB_D, do_block, 0)
        plsc.subcore_barrier()

        pltpu.sync_copy(dacc.at[pl.ds(nbase, ACC_PT)],
                        deg_out.at[pl.ds(c * N + nbase, ACC_PT)])

        @pl.when(s == NS - 1)
        def _():
            pltpu.sync_copy(dacc.at[pl.ds(tail0, N - tail0)],
                            deg_out.at[pl.ds(c * N + tail0, N - tail0)])

    return pl.kernel(body,
                     out_type=[jax.ShapeDtypeStruct((2 * N, H), jnp.float32)],
                     mesh=mesh, scratch_types=scratch)


@functools.lru_cache(maxsize=None)
def _sc_agg(edge_wide):
    return _make_sc_agg(edge_wide=edge_wide)


@functools.lru_cache(maxsize=None)
def _sc_deg():
    return _make_sc_deg()


def _dense_body(agg_ref, deg_ref, x_ref, wn_ref, ws_ref, h_ref, sums_ref):
    i = pl.program_id(0)
    aggf = jnp.concatenate([agg_ref[0], agg_ref[1]], axis=1)
    xf = jnp.concatenate([x_ref[0], x_ref[1]], axis=1)
    d = deg_ref[0][:, 0:1] + deg_ref[1][:, 0:1]
    r = 1.0 / jnp.maximum(d, 1.0)
    h = (jnp.dot(aggf * r, wn_ref[...], preferred_element_type=jnp.float32)
         + jnp.dot(xf, ws_ref[...], preferred_element_type=jnp.float32))
    h_ref[...] = h

    @pl.when(i == 0)
    def _():
        sums_ref[...] = jnp.zeros_like(sums_ref)

    sums_ref[0:1, :] += jnp.sum(h, axis=0, keepdims=True)
    sums_ref[1:2, :] += jnp.sum(h * h, axis=0, keepdims=True)


def _dense(agg_st, deg, x_st, wn, ws):
    return pl.pallas_call(
        _dense_body,
        grid=(N // BN,),
        in_specs=[
            pl.BlockSpec((2, BN, H), lambda i: (0, i, 0)),
            pl.BlockSpec((2, BN, H), lambda i: (0, i, 0)),
            pl.BlockSpec((2, BN, H), lambda i: (0, i, 0)),
            pl.BlockSpec((D, D), lambda i: (0, 0)),
            pl.BlockSpec((D, D), lambda i: (0, 0)),
        ],
        out_specs=[
            pl.BlockSpec((BN, D), lambda i: (i, 0)),
            pl.BlockSpec((8, D), lambda i: (0, 0)),
        ],
        out_shape=[
            jax.ShapeDtypeStruct((N, D), jnp.float32),
            jax.ShapeDtypeStruct((8, D), jnp.float32),
        ],
    )(agg_st, deg, x_st, wn, ws)


def _norm_body(h_ref, sums_ref, g_ref, b_ref, out_ref, *, stacked):
    m = sums_ref[0:1, :] / float(N)
    v = sums_ref[1:2, :] / float(N) - m * m
    inv = lax.rsqrt(v + 1e-5)
    y = (h_ref[...] - m) * (inv * g_ref[...]) + b_ref[...]
    y = jnp.maximum(y, 0.0)
    if stacked:
        out_ref[0] = y[:, :H]
        out_ref[1] = y[:, H:]
    else:
        out_ref[...] = y


def _norm(h, sums, g, b, stacked):
    if stacked:
        out_spec = pl.BlockSpec((2, BN, H), lambda i: (0, i, 0))
        out_shape = jax.ShapeDtypeStruct((2, N, H), jnp.float32)
    else:
        out_spec = pl.BlockSpec((BN, D), lambda i: (i, 0))
        out_shape = jax.ShapeDtypeStruct((N, D), jnp.float32)
    return pl.pallas_call(
        functools.partial(_norm_body, stacked=stacked),
        grid=(N // BN,),
        in_specs=[
            pl.BlockSpec((BN, D), lambda i: (i, 0)),
            pl.BlockSpec((8, D), lambda i: (0, 0)),
            pl.BlockSpec((1, D), lambda i: (0, 0)),
            pl.BlockSpec((1, D), lambda i: (0, 0)),
        ],
        out_specs=out_spec,
        out_shape=out_shape,
    )(h, sums, g, b)


def _edge_body(e_ref, w_ref, b_ref, out_ref):
    y = jnp.dot(e_ref[...], w_ref[...], preferred_element_type=jnp.float32)
    y = jnp.maximum(y + b_ref[...], 0.0)
    out_ref[0] = y[:, :H]
    out_ref[1] = y[:, H:]


def _edge_tf(edge, w, b):
    return pl.pallas_call(
        _edge_body,
        grid=(E // BE,),
        in_specs=[
            pl.BlockSpec((BE, D), lambda i: (i, 0)),
            pl.BlockSpec((D, D), lambda i: (0, 0)),
            pl.BlockSpec((1, D), lambda i: (0, 0)),
        ],
        out_specs=pl.BlockSpec((2, BE, H), lambda i: (0, i, 0)),
        out_shape=jax.ShapeDtypeStruct((2, E, H), jnp.float32),
    )(edge, w, b)


def _rel_body(rel_ref, w1_ref, b1_ref, w2_ref, b2_ref, out_ref):
    y = jnp.dot(rel_ref[...], w1_ref[...], preferred_element_type=jnp.float32)
    y = jnp.maximum(y + b1_ref[...], 0.0)
    y = jnp.dot(y, w2_ref[...], preferred_element_type=jnp.float32)
    out_ref[...] = jnp.maximum(y + b2_ref[...], 0.0)


def _rel_mlp(rel, w1, b1, w2, b2):
    return pl.pallas_call(
        _rel_body,
        out_shape=jax.ShapeDtypeStruct(rel.shape, jnp.float32),
    )(rel, w1, b1, w2, b2)


def kernel(node_embs, edge_embs, rel_embs, edge_index,
           W_n1, W_s1, g1, b1, W_e1, be1,
           W_n2, W_s2, g2, b2, W_e2, be2):
    src = edge_index[0]
    dst = edge_index[1]

    node_st = jnp.stack([node_embs[:, :H], node_embs[:, H:]])
    npad = ROWS_P * CH - E
    src_pad = jnp.concatenate([src, jnp.zeros((npad,), jnp.int32)])
    src2_r = jnp.concatenate([src_pad, src_pad + N]).reshape(
        2 * ROWS_P, NSUB, SUB)
    dst_pad = jnp.concatenate(
        [dst, N + (jnp.arange(npad, dtype=jnp.int32) % NTRASH)])
    dst_r = dst_pad.reshape(ROWS_P, NSUB, SUB)
    zacc = jnp.zeros((NA, H), jnp.float32)

    (deg2,) = _sc_deg()(dst_r, zacc)
    deg = deg2.reshape(2, N, H)
    e1r_st = _edge_tf(edge_embs, W_e1, be1.reshape(1, D))
    (agg1,) = _sc_agg(True)(node_st.reshape(2 * N, H), edge_embs,
                            src2_r, dst_r, zacc)
    h1, sums1 = _dense(agg1.reshape(2, N, H), deg, node_st, W_n1, W_s1)
    h1r_st = _norm(h1, sums1, g1.reshape(1, D), b1.reshape(1, D), stacked=True)

    (agg2,) = _sc_agg(False)(h1r_st.reshape(2 * N, H),
                             e1r_st.reshape(2 * E, H),
                             src2_r, dst_r, zacc)
    h2, sums2 = _dense(agg2.reshape(2, N, H), deg, h1r_st, W_n2, W_s2)
    nodes_out = _norm(h2, sums2, g2.reshape(1, D), b2.reshape(1, D),
                      stacked=False)

    r = _rel_mlp(rel_embs, W_e1, be1.reshape(1, D), W_e2, be2.reshape(1, D))
    return (nodes_out, r)
